# baseline (speedup 1.0000x reference)
import functools

import jax
import jax.numpy as jnp
from jax import lax
from jax.experimental import pallas as pl
from jax.experimental.pallas import tpu as pltpu
from jax.experimental.pallas import tpu_sc as plsc

B = 8
IN_C = 256
OUT_C = 80
D = 41
FH = 24
FW = 24
IH = 384
IW = 384
NX, NY, NZ = 200, 200, 1

HW = FH * FW
NPB = D * HW
CROP = 50
NYC = NY - CROP
XHALF = NX // 2
ROWS_HALF = XHALF * NYC
TRASH = 512
ACC_ROWS = ROWS_HALF + TRASH
ROWS_PER_SUB = 1000
HWT = HW // 16
DP = 48
MWORDS = HWT * DP
MROWS = 16
FWORDS = HWT * OUT_C
FROWS = 23


def _conv(x, w, b, stride, pad):
    o = jax.lax.conv_general_dilated(x, w, (stride, stride), ((pad, pad), (pad, pad)), dimension_numbers=('NCHW', 'OIHW', 'NCHW'))
    return o + b[None, :, None, None]


def _bn(x, g, be):
    m = x.mean(axis=(0, 2, 3), keepdims=True)
    v = x.var(axis=(0, 2, 3), keepdims=True)
    return g[None, :, None, None] * (x - m) / jnp.sqrt(v + 1e-5) + be[None, :, None, None]


def _frustum():
    ds = jnp.arange(4.0, 45.0, 1.0).reshape(-1, 1, 1) * jnp.ones((1, FH, FW), jnp.float32)
    xs = jnp.linspace(0.0, IW - 1.0, FW).reshape(1, 1, FW) * jnp.ones((D, FH, 1), jnp.float32)
    ys = jnp.linspace(0.0, IH - 1.0, FH).reshape(1, FH, 1) * jnp.ones((D, 1, FW), jnp.float32)
    return jnp.stack([xs, ys, ds], -1)


def _dtransform(d, p):
    x = jax.nn.relu(_bn(_conv(d, p['dt_w1'], p['dt_b1'], 1, 0), p['dt_g1'], p['dt_be1']))
    x = jax.nn.relu(_bn(_conv(x, p['dt_w2'], p['dt_b2'], 4, 2), p['dt_g2'], p['dt_be2']))
    x = jax.nn.relu(_bn(_conv(x, p['dt_w3'], p['dt_b3'], 2, 2), p['dt_g3'], p['dt_be3']))
    x = jax.nn.relu(_bn(_conv(x, p['dt_w4'], p['dt_b4'], 2, 2), p['dt_g4'], p['dt_be4']))
    return x


def _depthnet(x, p):
    x = jax.nn.relu(_bn(_conv(x, p['dn_w1'], p['dn_b1'], 1, 1), p['dn_g1'], p['dn_be1']))
    x = jax.nn.relu(_bn(_conv(x, p['dn_w2'], p['dn_b2'], 1, 1), p['dn_g2'], p['dn_be2']))
    x = _conv(x, p['dn_w3'], p['dn_b3'], 1, 0)
    return x


def _splat_body(feat_hbm, wdep_hbm, gxf_hbm, gyf_hbm, gzf_hbm, out_hbm,
                acc, feat_v, pay_v, w_v, gx_v, gy_v, gz_v, idx_v, vbufa, vbufb,
                mstage, fstage, zbuf):
    c = lax.axis_index("c")
    s = lax.axis_index("s")
    cx0 = c * XHALF
    lanes = lax.iota(jnp.int32, 16)

    for o in range(0, 3200, 16):
        zbuf[o // 80, pl.ds(o % 80, 16)] = jnp.zeros((16,), jnp.float32)

    def batch_body(b, carry):
        @pl.when((b == 0) & (s < 15))
        def _zero0():
            def z_body(q, cz):
                pltpu.sync_copy(zbuf, acc.at[pl.ds(s * ROWS_PER_SUB + q * 40, 40)])
                return cz

            lax.fori_loop(0, ROWS_PER_SUB // 40, z_body, 0)

        def stage_meta(hbm, dst):
            pltpu.sync_copy(hbm.at[pl.ds((b * 16 + s) * MROWS, MROWS)], mstage)
            for o in range(0, MWORDS, 16):
                dst[pl.ds(o, 16)] = mstage[o // 128, pl.ds(o % 128, 16)]

        pltpu.sync_copy(feat_hbm.at[pl.ds((b * 16 + s) * FROWS, FROWS)], fstage)
        for o in range(0, FWORDS, 16):
            feat_v[pl.ds(o, 16)] = fstage[o // 128, pl.ds(o % 128, 16)]
        stage_meta(wdep_hbm, w_v)
        stage_meta(gxf_hbm, gx_v)
        stage_meta(gyf_hbm, gy_v)
        stage_meta(gzf_hbm, gz_v)
        plsc.subcore_barrier()

        def pair_body(t, carry2):
            for u in range(2):
                hwl = 2 * t + u
                fk = [feat_v[pl.ds(hwl * OUT_C + k * 16, 16)] for k in range(5)]
                for g in range(3):
                    msl = pl.ds(hwl * DP + g * 16, 16)
                    fx = (gx_v[msl] + 50.0) / 0.5
                    fy = (gy_v[msl] + 50.0) / 0.5
                    fz = (gz_v[msl] + 10.0) / 20.0
                    okf = ((fx > -3e4) & (fx < 3e4) & (fy > -3e4) & (fy < 3e4)
                           & (fz > -3e4) & (fz < 3e4))
                    ix = jnp.where(okf, fx, -3e4).astype(jnp.int32)
                    iy = jnp.where(okf, fy, -3e4).astype(jnp.int32)
                    iz = jnp.where(okf, fz, -3e4).astype(jnp.int32)
                    keep = ((ix >= cx0) & (ix < cx0 + XHALF)
                            & (iy >= CROP) & (iy < NY) & (iz == 0))
                    widx = (ix - cx0) * NYC + (iy - CROP)
                    trash = ROWS_HALF + ((g * 16 + s * 89 + lanes) & (TRASH - 1))
                    idx_v[0, pl.ds(u * DP + g * 16, 16)] = jnp.where(keep, widx, trash)
                    wv = w_v[pl.ds(hwl * DP + g * 16, 16)]
                    for l in range(16):
                        row = u * DP + g * 16 + l
                        w = wv[l]
                        for k in range(5):
                            pay_v[row, pl.ds(k * 16, 16)] = w * fk[k]
            pltpu.sync_copy(pay_v, acc.at[idx_v.at[0]], add=True)
            return carry2

        lax.fori_loop(0, HWT // 2, pair_body, 0)
        plsc.subcore_barrier()

        @pl.when(s < 15)
        def _readout():
            def rd_body(q, carry3):
                pltpu.sync_copy(acc.at[pl.ds(s * ROWS_PER_SUB + q * 40, 40)], vbufa)
                for o in range(0, 3200, 16):
                    vbufb[o // 128, pl.ds(o % 128, 16)] = vbufa[o // 80, pl.ds(o % 80, 16)]
                blk = ((b * 2 + c) * (ROWS_HALF * OUT_C // 128)
                       + s * (ROWS_PER_SUB * OUT_C // 128) + q * 25)
                pltpu.sync_copy(vbufb, out_hbm.at[pl.ds(blk, 25)])
                pltpu.sync_copy(zbuf, acc.at[pl.ds(s * ROWS_PER_SUB + q * 40, 40)])
                return carry3

            lax.fori_loop(0, ROWS_PER_SUB // 40, rd_body, 0)

        plsc.subcore_barrier()
        return carry

    lax.fori_loop(0, B, batch_body, 0)


_SPLAT_CACHE = {}


def _get_splat_kernel():
    if "k" not in _SPLAT_CACHE:
        _SPLAT_CACHE["k"] = pl.kernel(
            _splat_body,
            out_type=jax.ShapeDtypeStruct((B * 2 * ROWS_HALF * OUT_C // 128, 128), jnp.float32),
            mesh=plsc.VectorSubcoreMesh(core_axis_name="c", subcore_axis_name="s",
                                        num_cores=2, num_subcores=16),
            scratch_types=[
                pltpu.VMEM_SHARED((ACC_ROWS, OUT_C), jnp.float32),
                pltpu.VMEM((HWT * OUT_C,), jnp.float32),
                pltpu.VMEM((2 * DP, OUT_C), jnp.float32),
                pltpu.VMEM((HWT * DP,), jnp.float32),
                pltpu.VMEM((HWT * DP,), jnp.float32),
                pltpu.VMEM((HWT * DP,), jnp.float32),
                pltpu.VMEM((HWT * DP,), jnp.float32),
                pltpu.VMEM((1, 2 * DP), jnp.int32),
                pltpu.VMEM((40, OUT_C), jnp.float32),
                pltpu.VMEM((25, 128), jnp.float32),
                pltpu.VMEM((MROWS, 128), jnp.float32),
                pltpu.VMEM((FROWS, 128), jnp.float32),
                pltpu.VMEM((40, OUT_C), jnp.float32),
            ],
            compiler_params=pltpu.CompilerParams(use_tc_tiling_on_sc=False),
        )
    return _SPLAT_CACHE["k"]


def kernel(x_feat, intrins, depth_x, params):
    ds = jnp.arange(4.0, 45.0, 1.0)
    xs = jnp.linspace(0.0, IW - 1.0, FW)
    ys = jnp.linspace(0.0, IH - 1.0, FH)
    pts = jnp.stack([
        xs[None, :, None] * ds[None, None, :] * jnp.ones((FH, 1, 1), jnp.float32),
        ys[:, None, None] * ds[None, None, :] * jnp.ones((1, FW, 1), jnp.float32),
        jnp.broadcast_to(ds[None, None, :], (FH, FW, D)),
    ], axis=-1)
    rots = jnp.array([[1.0, 0.0, 0.0], [0.0, 0.0, 1.0], [0.0, -1.0, 0.0]], jnp.float32)
    combine = jnp.matmul(rots[None], jnp.linalg.inv(intrins.reshape(B, 3, 3)))
    pad3 = ((0, 0), (0, 0), (0, 0), (0, DP - D))
    def metachunks(x):
        x = x.reshape(B, 16, MWORDS)
        x = jnp.pad(x, ((0, 0), (0, 0), (0, MROWS * 128 - MWORDS)))
        return x.reshape(B * 16 * MROWS, 128)

    gxf = metachunks(jnp.pad(jnp.einsum('bj,hwdj->bhwd', combine[:, 0, :], pts),
                             pad3, constant_values=1e9))
    gyf = metachunks(jnp.pad(jnp.einsum('bj,hwdj->bhwd', combine[:, 1, :], pts),
                             pad3, constant_values=1e9))
    gzf = metachunks(jnp.pad(jnp.einsum('bj,hwdj->bhwd', combine[:, 2, :], pts),
                             pad3, constant_values=1e9))
    x = x_feat.reshape(B, IN_C, FH, FW)
    _d = _dtransform(depth_x, params)
    x = jnp.concatenate([_d, x], axis=1)
    x = jax.nn.relu(_bn(_conv(x, params['dn_w1'], params['dn_b1'], 1, 1), params['dn_g1'], params['dn_be1']))
    x = jax.nn.relu(_bn(_conv(x, params['dn_w2'], params['dn_b2'], 1, 1), params['dn_g2'], params['dn_be2']))
    feat = (jnp.einsum('bchw,oc->bhwo', x, params['dn_w3'][:, :, 0, 0])
            + params['dn_b3'][None, None, None, :]).reshape(B, 16, FWORDS)
    feat = jnp.pad(feat, ((0, 0), (0, 0), (0, FROWS * 128 - FWORDS)))
    feat = feat.reshape(B * 16 * FROWS, 128)
    avgd = depth_x.reshape(B, FH, 16, FW, 16).mean(axis=(2, 4))
    mean_d = jnp.floor(avgd / 1000.0 * D)[..., None]
    xg = jnp.arange(DP, dtype=jnp.float32)
    sigma = 0.5
    gauss = (1.0 / jnp.sqrt(2.0 * jnp.pi)) * sigma * jnp.exp(-(xg - mean_d) ** 2 / 2.0 * sigma ** 2)
    gauss = jnp.where(xg < D, gauss, -jnp.inf)
    wdep = metachunks(jax.nn.softmax(gauss, axis=-1))
    out = _get_splat_kernel()(feat, wdep, gxf, gyf, gzf)
    out = out.reshape(B, 2, XHALF, NYC, OUT_C).transpose(0, 4, 1, 2, 3)
    out = out.reshape(B, OUT_C, NX, NYC)
    neg = intrins[0, 0, 2, 2] < -1e30
    return jnp.where(neg, jnp.float32(0), out)


# speedup vs baseline: 1.3338x; 1.3338x over previous
import functools

import jax
import jax.numpy as jnp
from jax import lax
from jax.experimental import pallas as pl
from jax.experimental.pallas import tpu as pltpu
from jax.experimental.pallas import tpu_sc as plsc

B = 8
IN_C = 256
OUT_C = 80
D = 41
FH = 24
FW = 24
IH = 384
IW = 384
NX, NY, NZ = 200, 200, 1

HW = FH * FW
NPB = D * HW
CROP = 50
NYC = NY - CROP
XHALF = NX // 2
ROWS_HALF = XHALF * NYC
TRASH = 512
ACC_ROWS = ROWS_HALF + TRASH
ROWS_PER_SUB = 1000
HWT = HW // 16
DP = 48
MWORDS = HWT * DP
MROWS = 16
FWORDS = HWT * OUT_C
FROWS = 23


def _conv(x, w, b, stride, pad):
    o = jax.lax.conv_general_dilated(x, w, (stride, stride), ((pad, pad), (pad, pad)), dimension_numbers=('NCHW', 'OIHW', 'NCHW'))
    return o + b[None, :, None, None]


def _bn(x, g, be):
    m = x.mean(axis=(0, 2, 3), keepdims=True)
    v = x.var(axis=(0, 2, 3), keepdims=True)
    return g[None, :, None, None] * (x - m) / jnp.sqrt(v + 1e-5) + be[None, :, None, None]


def _frustum():
    ds = jnp.arange(4.0, 45.0, 1.0).reshape(-1, 1, 1) * jnp.ones((1, FH, FW), jnp.float32)
    xs = jnp.linspace(0.0, IW - 1.0, FW).reshape(1, 1, FW) * jnp.ones((D, FH, 1), jnp.float32)
    ys = jnp.linspace(0.0, IH - 1.0, FH).reshape(1, FH, 1) * jnp.ones((D, 1, FW), jnp.float32)
    return jnp.stack([xs, ys, ds], -1)


def _dtransform(d, p):
    x0 = d * p['dt_w1'][None, :, 0, 0, None] + p['dt_b1'][None, :, None, None]
    x = jax.nn.relu(_bn(x0, p['dt_g1'], p['dt_be1']))
    x = jax.nn.relu(_bn(_conv(x, p['dt_w2'], p['dt_b2'], 4, 2), p['dt_g2'], p['dt_be2']))
    x = jax.nn.relu(_bn(_conv(x, p['dt_w3'], p['dt_b3'], 2, 2), p['dt_g3'], p['dt_be3']))
    x = jax.nn.relu(_bn(_conv(x, p['dt_w4'], p['dt_b4'], 2, 2), p['dt_g4'], p['dt_be4']))
    return x


def _depthnet(x, p):
    x = jax.nn.relu(_bn(_conv(x, p['dn_w1'], p['dn_b1'], 1, 1), p['dn_g1'], p['dn_be1']))
    x = jax.nn.relu(_bn(_conv(x, p['dn_w2'], p['dn_b2'], 1, 1), p['dn_g2'], p['dn_be2']))
    x = _conv(x, p['dn_w3'], p['dn_b3'], 1, 0)
    return x


def _splat_body(feat_hbm, wdep_hbm, gxf_hbm, gyf_hbm, gzf_hbm, out_hbm,
                acc, feat_v, pay_v, w_v, gx_v, gy_v, gz_v, idx_v, vbufa, vbufb,
                mstage, fstage, zbuf):
    c = lax.axis_index("c")
    s = lax.axis_index("s")
    cx0 = c * XHALF
    lanes = lax.iota(jnp.int32, 16)

    for o in range(0, 3200, 16):
        zbuf[o // 80, pl.ds(o % 80, 16)] = jnp.zeros((16,), jnp.float32)

    def batch_body(b, carry):
        @pl.when((b == 0) & (s < 15))
        def _zero0():
            def z_body(q, cz):
                pltpu.sync_copy(zbuf, acc.at[pl.ds(s * ROWS_PER_SUB + q * 40, 40)])
                return cz

            lax.fori_loop(0, ROWS_PER_SUB // 40, z_body, 0)

        def stage_meta(hbm, dst):
            pltpu.sync_copy(hbm.at[pl.ds((b * 16 + s) * MROWS, MROWS)], mstage)
            for o in range(0, MWORDS, 16):
                dst[pl.ds(o, 16)] = mstage[o // 128, pl.ds(o % 128, 16)]

        pltpu.sync_copy(feat_hbm.at[pl.ds((b * 16 + s) * FROWS, FROWS)], fstage)
        for o in range(0, FWORDS, 16):
            feat_v[pl.ds(o, 16)] = fstage[o // 128, pl.ds(o % 128, 16)]
        stage_meta(wdep_hbm, w_v)
        stage_meta(gxf_hbm, gx_v)
        stage_meta(gyf_hbm, gy_v)
        stage_meta(gzf_hbm, gz_v)
        plsc.subcore_barrier()

        def pair_body(t, carry2):
            for u in range(2):
                hwl = 2 * t + u
                fk = [feat_v[pl.ds(hwl * OUT_C + k * 16, 16)] for k in range(5)]
                for g in range(3):
                    msl = pl.ds(hwl * DP + g * 16, 16)
                    fx = (gx_v[msl] + 50.0) / 0.5
                    fy = (gy_v[msl] + 50.0) / 0.5
                    fz = (gz_v[msl] + 10.0) / 20.0
                    okf = ((fx > -3e4) & (fx < 3e4) & (fy > -3e4) & (fy < 3e4)
                           & (fz > -3e4) & (fz < 3e4))
                    ix = jnp.where(okf, fx, -3e4).astype(jnp.int32)
                    iy = jnp.where(okf, fy, -3e4).astype(jnp.int32)
                    iz = jnp.where(okf, fz, -3e4).astype(jnp.int32)
                    keep = ((ix >= cx0) & (ix < cx0 + XHALF)
                            & (iy >= CROP) & (iy < NY) & (iz == 0))
                    widx = (ix - cx0) * NYC + (iy - CROP)
                    trash = ROWS_HALF + ((g * 16 + s * 89 + lanes) & (TRASH - 1))
                    idx_v[0, pl.ds(u * DP + g * 16, 16)] = jnp.where(keep, widx, trash)
                    wv = w_v[pl.ds(hwl * DP + g * 16, 16)]
                    for l in range(16):
                        row = u * DP + g * 16 + l
                        w = wv[l]
                        for k in range(5):
                            pay_v[row, pl.ds(k * 16, 16)] = w * fk[k]
            pltpu.sync_copy(pay_v, acc.at[idx_v.at[0]], add=True)
            return carry2

        lax.fori_loop(0, HWT // 2, pair_body, 0)
        plsc.subcore_barrier()

        @pl.when(s < 15)
        def _readout():
            def rd_body(q, carry3):
                pltpu.sync_copy(acc.at[pl.ds(s * ROWS_PER_SUB + q * 40, 40)], vbufa)
                for o in range(0, 3200, 16):
                    vbufb[o // 128, pl.ds(o % 128, 16)] = vbufa[o // 80, pl.ds(o % 80, 16)]
                blk = ((b * 2 + c) * (ROWS_HALF * OUT_C // 128)
                       + s * (ROWS_PER_SUB * OUT_C // 128) + q * 25)
                pltpu.sync_copy(vbufb, out_hbm.at[pl.ds(blk, 25)])
                pltpu.sync_copy(zbuf, acc.at[pl.ds(s * ROWS_PER_SUB + q * 40, 40)])
                return carry3

            lax.fori_loop(0, ROWS_PER_SUB // 40, rd_body, 0)

        plsc.subcore_barrier()
        return carry

    lax.fori_loop(0, B, batch_body, 0)


_SPLAT_CACHE = {}


def _get_splat_kernel():
    if "k" not in _SPLAT_CACHE:
        _SPLAT_CACHE["k"] = pl.kernel(
            _splat_body,
            out_type=jax.ShapeDtypeStruct((B * 2 * ROWS_HALF * OUT_C // 128, 128), jnp.float32),
            mesh=plsc.VectorSubcoreMesh(core_axis_name="c", subcore_axis_name="s",
                                        num_cores=2, num_subcores=16),
            scratch_types=[
                pltpu.VMEM_SHARED((ACC_ROWS, OUT_C), jnp.float32),
                pltpu.VMEM((HWT * OUT_C,), jnp.float32),
                pltpu.VMEM((2 * DP, OUT_C), jnp.float32),
                pltpu.VMEM((HWT * DP,), jnp.float32),
                pltpu.VMEM((HWT * DP,), jnp.float32),
                pltpu.VMEM((HWT * DP,), jnp.float32),
                pltpu.VMEM((HWT * DP,), jnp.float32),
                pltpu.VMEM((1, 2 * DP), jnp.int32),
                pltpu.VMEM((40, OUT_C), jnp.float32),
                pltpu.VMEM((25, 128), jnp.float32),
                pltpu.VMEM((MROWS, 128), jnp.float32),
                pltpu.VMEM((FROWS, 128), jnp.float32),
                pltpu.VMEM((40, OUT_C), jnp.float32),
            ],
            compiler_params=pltpu.CompilerParams(use_tc_tiling_on_sc=False),
        )
    return _SPLAT_CACHE["k"]


def kernel(x_feat, intrins, depth_x, params):
    ds = jnp.arange(4.0, 45.0, 1.0)
    xs = jnp.linspace(0.0, IW - 1.0, FW)
    ys = jnp.linspace(0.0, IH - 1.0, FH)
    pts = jnp.stack([
        xs[None, :, None] * ds[None, None, :] * jnp.ones((FH, 1, 1), jnp.float32),
        ys[:, None, None] * ds[None, None, :] * jnp.ones((1, FW, 1), jnp.float32),
        jnp.broadcast_to(ds[None, None, :], (FH, FW, D)),
    ], axis=-1)
    rots = jnp.array([[1.0, 0.0, 0.0], [0.0, 0.0, 1.0], [0.0, -1.0, 0.0]], jnp.float32)
    combine = jnp.matmul(rots[None], jnp.linalg.inv(intrins.reshape(B, 3, 3)))
    pad3 = ((0, 0), (0, 0), (0, 0), (0, DP - D))
    def metachunks(x):
        x = x.reshape(B, 16, MWORDS)
        x = jnp.pad(x, ((0, 0), (0, 0), (0, MROWS * 128 - MWORDS)))
        return x.reshape(B * 16 * MROWS, 128)

    gxf = metachunks(jnp.pad(jnp.einsum('bj,hwdj->bhwd', combine[:, 0, :], pts),
                             pad3, constant_values=1e9))
    gyf = metachunks(jnp.pad(jnp.einsum('bj,hwdj->bhwd', combine[:, 1, :], pts),
                             pad3, constant_values=1e9))
    gzf = metachunks(jnp.pad(jnp.einsum('bj,hwdj->bhwd', combine[:, 2, :], pts),
                             pad3, constant_values=1e9))
    x = x_feat.reshape(B, IN_C, FH, FW)
    _d = _dtransform(depth_x, params)
    x = jnp.concatenate([_d, x], axis=1)
    x = jax.nn.relu(_bn(_conv(x, params['dn_w1'], params['dn_b1'], 1, 1), params['dn_g1'], params['dn_be1']))
    x = jax.nn.relu(_bn(_conv(x, params['dn_w2'], params['dn_b2'], 1, 1), params['dn_g2'], params['dn_be2']))
    feat = (jnp.einsum('bchw,oc->bhwo', x, params['dn_w3'][:, :, 0, 0])
            + params['dn_b3'][None, None, None, :]).reshape(B, 16, FWORDS)
    feat = jnp.pad(feat, ((0, 0), (0, 0), (0, FROWS * 128 - FWORDS)))
    feat = feat.reshape(B * 16 * FROWS, 128)
    avgd = depth_x.reshape(B, FH, 16, FW, 16).mean(axis=(2, 4))
    mean_d = jnp.floor(avgd / 1000.0 * D)[..., None]
    xg = jnp.arange(DP, dtype=jnp.float32)
    sigma = 0.5
    gauss = (1.0 / jnp.sqrt(2.0 * jnp.pi)) * sigma * jnp.exp(-(xg - mean_d) ** 2 / 2.0 * sigma ** 2)
    gauss = jnp.where(xg < D, gauss, -jnp.inf)
    wdep = metachunks(jax.nn.softmax(gauss, axis=-1))
    out = _get_splat_kernel()(feat, wdep, gxf, gyf, gzf)
    out = out.reshape(B, 2, XHALF, NYC, OUT_C).transpose(0, 4, 1, 2, 3)
    out = out.reshape(B, OUT_C, NX, NYC)
    neg = intrins[0, 0, 2, 2] < -1e30
    return jnp.where(neg, jnp.float32(0), out)
